# baseline (speedup 1.0000x reference)
import functools

import jax
import jax.numpy as jnp
from jax import lax
from jax.experimental import pallas as pl
from jax.experimental.pallas import tpu as pltpu
from jax.experimental.pallas import tpu_sc as plsc

N = 2
HW = 4097
HWP = 4104
D = 768
K = 1024
L = 16
NW = 32
G = N * D // L
GPW = G // NW
R = HWP // 8
NB = 256


def _topk_body(x6_hbm, out_hbm, tslab, hist):
    wid = lax.axis_index("s") * 2 + lax.axis_index("c")

    lane = lax.iota(jnp.int32, L)
    ones = jnp.ones((L,), jnp.int32)
    mones = jnp.full((L,), -1, jnp.int32)
    zeros = jnp.zeros((L,), jnp.int32)
    zf = jnp.zeros((L,), jnp.float32)
    kfull = jnp.full((L,), K, jnp.int32)

    @plsc.parallel_loop(0, NB, unroll=8)
    def _(b):
        hist[pl.ds(b * L, L)] = zeros

    for s in range(GPW):
        g = wid * GPW + s
        pltpu.sync_copy(x6_hbm.at[g], tslab)

        v0 = tslab[0, pl.ds(0, L)]
        bits0 = plsc.bitcast(v0, jnp.int32) & 0x7FFFFFFF

        kk = kfull
        t_prefix = zeros
        for p in range(3):
            shift = 23 - 8 * p

            if p == 0:
                @plsc.parallel_loop(0, R, unroll=2)
                def _(r):
                    for m in range(8):
                        bits = plsc.bitcast(tslab[r, pl.ds(m * L, L)],
                                            jnp.int32) & 0x7FFFFFFF
                        b = bits >> shift
                        plsc.addupdate_scatter(hist, [b * L + lane], ones)

                plsc.addupdate_scatter(hist, [(bits0 >> shift) * L + lane],
                                       mones)
            else:
                @plsc.parallel_loop(0, R, unroll=2)
                def _(r, shift=shift, t_prefix=t_prefix):
                    for m in range(8):
                        bits = plsc.bitcast(tslab[r, pl.ds(m * L, L)],
                                            jnp.int32) & 0x7FFFFFFF
                        b = (bits >> shift) & 0xFF
                        msk = (bits >> (shift + 8)) == t_prefix
                        plsc.addupdate_scatter(hist, [b * L + lane], ones,
                                               mask=msk)

                plsc.addupdate_scatter(
                    hist, [((bits0 >> shift) & 0xFF) * L + lane], mones,
                    mask=(bits0 >> (shift + 8)) == t_prefix)

            @plsc.parallel_loop(0, NB, unroll=4, carry=(zeros, zeros, zeros))
            def scan_out(j, carry):
                acc, bsel, above = carry
                b = NB - 1 - j
                h = hist[pl.ds(b * L, L)]
                hist[pl.ds(b * L, L)] = zeros
                acc2 = acc + h
                crossed = (acc < kk) & (acc2 >= kk)
                bsel = jnp.where(crossed, zeros + b, bsel)
                above = jnp.where(crossed, acc, above)
                return (acc2, bsel, above)

            _, bsel, above = scan_out
            kk = kk - above
            t_prefix = (t_prefix << 8) | bsel

        t24 = t_prefix

        @plsc.parallel_loop(0, R, unroll=2)
        def _(r):
            for m in range(8):
                v = tslab[r, pl.ds(m * L, L)]
                bits = plsc.bitcast(v, jnp.int32) & 0x7FFFFFFF
                tslab[r, pl.ds(m * L, L)] = jnp.where((bits >> 7) >= t24,
                                                      v, zf)

        tslab[0, pl.ds(0, L)] = v0
        pltpu.sync_copy(tslab, out_hbm.at[g])


_topk_call = functools.partial(
    pl.kernel,
    out_type=jax.ShapeDtypeStruct((G, R, 128), jnp.float32),
    mesh=plsc.VectorSubcoreMesh(core_axis_name="c", subcore_axis_name="s"),
    scratch_types=[
        pltpu.VMEM((R, 128), jnp.float32),
        pltpu.VMEM((NB * L,), jnp.int32),
    ],
    compiler_params=pltpu.CompilerParams(needs_layout_passes=False),
)(_topk_body)


@jax.jit
def kernel(x):
    x6 = jnp.pad(x, ((0, 0), (0, HWP - HW), (0, 0)))
    x6 = x6.reshape(N, HWP, D // L, L).transpose(0, 2, 1, 3).reshape(G, R, 128)
    y6 = _topk_call(x6)
    y = y6.reshape(N, D // L, HWP, L).transpose(0, 2, 1, 3)[:, :HW]
    return y.reshape(N, HW, D)


# speedup vs baseline: 4.3098x; 4.3098x over previous
import functools

import jax
import jax.numpy as jnp
from jax import lax
from jax.experimental import pallas as pl
from jax.experimental.pallas import tpu as pltpu
from jax.experimental.pallas import tpu_sc as plsc

N = 2
HW = 4097
D = 768
K = 1024
L = 16
NC = 2
NS = 16
CH_PER_W = (N * D) // (NC * NS)
NSLAB = CH_PER_W // L
NB = 256
HALF = 2048


def _topk_body(xt_hbm, out_hbm, cbuf, tslab, hist, sliver):
    wid = lax.axis_index("s") * NC + lax.axis_index("c")
    n = wid // (D // CH_PER_W)
    cbase = (wid % (D // CH_PER_W)) * CH_PER_W

    lane = lax.iota(jnp.int32, L)
    lane16 = lane * 16
    ones = jnp.ones((L,), jnp.int32)
    zeros = jnp.zeros((L,), jnp.int32)
    zf = jnp.zeros((L,), jnp.float32)
    kfull = jnp.full((L,), K, jnp.int32)

    @plsc.parallel_loop(0, NB, unroll=8)
    def _(b):
        hist[pl.ds(b * L, L)] = zeros

    def slab_body(s, _):
        c0 = cbase + s * L

        for h in range(2):
            pltpu.sync_copy(xt_hbm.at[n, pl.ds(c0, L), pl.ds(h * HALF, HALF)],
                            cbuf)
            for c in range(L):
                skew_c = lane16 + ((lane + c) & 15)
                base0 = h * HALF * L

                @plsc.parallel_loop(0, HALF // L, unroll=8)
                def _(j, c=c, skew_c=skew_c, base0=base0):
                    v = cbuf[c, pl.ds(j * L, L)]
                    idx = skew_c + (base0 + j * L * L)
                    plsc.store_scatter(tslab, [idx], v)

        pltpu.sync_copy(xt_hbm.at[n, pl.ds(c0, L), pl.ds(HW - 1, 1)], sliver)
        vlast = plsc.load_gather(sliver, [lane, zeros])
        plsc.store_scatter(tslab, [lane + (HW - 1) * L], vlast)

        kk = kfull
        t_prefix = zeros
        for p in range(3):
            shift = 23 - 8 * p

            if p == 0:
                @plsc.parallel_loop(1, HW, unroll=8)
                def _(i):
                    bits = plsc.bitcast(tslab[pl.ds(i * L, L)],
                                        jnp.int32) & 0x7FFFFFFF
                    b = bits >> shift
                    cvec = (lane - i) & 15
                    plsc.addupdate_scatter(hist, [b * L + cvec], ones)
            else:
                @plsc.parallel_loop(1, HW, unroll=8)
                def _(i, shift=shift, t_prefix=t_prefix):
                    bits = plsc.bitcast(tslab[pl.ds(i * L, L)],
                                        jnp.int32) & 0x7FFFFFFF
                    b = (bits >> shift) & 0xFF
                    cvec = (lane - i) & 15
                    m = (bits >> (shift + 8)) == t_prefix.at[cvec].get(
                        mode="promise_in_bounds")
                    plsc.addupdate_scatter(hist, [b * L + cvec], ones, mask=m)

            @plsc.parallel_loop(0, NB, unroll=4, carry=(zeros, zeros, zeros))
            def scan_out(j, carry):
                acc, bsel, above = carry
                b = NB - 1 - j
                h = hist[pl.ds(b * L, L)]
                hist[pl.ds(b * L, L)] = zeros
                acc2 = acc + h
                crossed = (acc < kk) & (acc2 >= kk)
                bsel = jnp.where(crossed, zeros + b, bsel)
                above = jnp.where(crossed, acc, above)
                return (acc2, bsel, above)

            _, bsel, above = scan_out
            kk = kk - above
            t_prefix = (t_prefix << 8) | bsel

        for h in range(2):
            for c in range(L):
                tvec = jnp.full((L,), 0, jnp.int32) + t_prefix[c]
                cvec = zeros + c
                skew_c = lane16 + ((lane + c) & 15)
                base0 = h * HALF * L

                @plsc.parallel_loop(0, HALF // L, unroll=8)
                def _(j, c=c, skew_c=skew_c, base0=base0, tvec=tvec,
                      cvec=cvec):
                    idx = skew_c + (base0 + j * L * L)
                    v = plsc.load_gather(tslab, [idx])
                    bits = plsc.bitcast(v, jnp.int32) & 0x7FFFFFFF
                    keep = ((bits >> 7) >= tvec) | (idx == cvec)
                    cbuf[c, pl.ds(j * L, L)] = jnp.where(keep, v, zf)

            pltpu.sync_copy(cbuf,
                            out_hbm.at[n, pl.ds(c0, L), pl.ds(h * HALF, HALF)])

        vlast = plsc.load_gather(tslab, [lane + (HW - 1) * L])
        lbits = plsc.bitcast(vlast, jnp.int32) & 0x7FFFFFFF
        vmasked = jnp.where((lbits >> 7) >= t_prefix, vlast, zf)
        plsc.store_scatter(sliver, [lane, zeros], vmasked)
        pltpu.sync_copy(sliver, out_hbm.at[n, pl.ds(c0, L), pl.ds(HW - 1, 1)])
        return 0

    lax.fori_loop(0, NSLAB, slab_body, 0)


_topk_call = functools.partial(
    pl.kernel,
    out_type=jax.ShapeDtypeStruct((N, D, HW), jnp.float32),
    mesh=plsc.VectorSubcoreMesh(core_axis_name="c", subcore_axis_name="s"),
    scratch_types=[
        pltpu.VMEM((L, HALF), jnp.float32),
        pltpu.VMEM((HW * L,), jnp.float32),
        pltpu.VMEM((NB * L,), jnp.int32),
        pltpu.VMEM((L, 1), jnp.float32),
    ],
    compiler_params=pltpu.CompilerParams(needs_layout_passes=False),
)(_topk_body)


@jax.jit
def kernel(x):
    xt = jnp.transpose(x, (0, 2, 1))
    yt = _topk_call(xt)
    return jnp.transpose(yt, (0, 2, 1))
